# baseline (speedup 1.0000x reference)
import jax
import jax.numpy as jnp
from jax import lax
from jax.experimental import pallas as pl
from jax.experimental.pallas import tpu as pltpu

SRC_VOCAB = 100000
TGT_VOCAB = 100000
D = 64
B, S_SRC, S_TGT = 32, 200, 16
N_SRC = B * S_SRC
N_TGT = B * S_TGT
V_TILE = 8192


def _gatt_body(goff_ref, tidx_ref, stab_ref, oh_ref, ttab_ref, out_ref,
               se_buf, te_buf, gbuf, sem):
    def issue_t(i, c):
        pltpu.make_async_copy(ttab_ref.at[pl.ds(tidx_ref[i], 1)],
                              te_buf.at[pl.ds(i, 1)], sem).start()
        return c

    lax.fori_loop(0, N_TGT, issue_t, 0, unroll=8)

    def pick8(i, c):
        for j in range(8):
            gbuf[pl.ds(j * 8, 8), :] = stab_ref[pl.ds(goff_ref[8 * i + j], 8), :]
        sel = lax.dot_general(oh_ref[pl.ds(8 * i, 8), :], gbuf[...],
                              (((1,), (0,)), ((), ())),
                              preferred_element_type=jnp.float32)
        se_buf[pl.ds(8 * i, 8), :] = sel
        return c

    lax.fori_loop(0, N_SRC // 8, pick8, 0)

    pltpu.make_async_copy(ttab_ref.at[pl.ds(0, N_TGT)], te_buf, sem).wait()

    for b in range(B):
        se_b = se_buf[pl.ds(b * S_SRC, S_SRC), :]
        te_b = te_buf[pl.ds(b * S_TGT, S_TGT), :]
        s = lax.dot_general(te_b, se_b, (((1,), (1,)), ((), ())),
                            preferred_element_type=jnp.float32) * 0.125
        s = s - jnp.max(s, axis=1, keepdims=True)
        e = jnp.exp(s)
        a = e / jnp.sum(e, axis=1, keepdims=True)
        o = lax.dot_general(a, se_b, (((1,), (0,)), ((), ())),
                            preferred_element_type=jnp.float32)
        out_ref[:, b, :] = o


def _proj_body(a_ref, w_ref, b_ref, out_ref):
    out = lax.dot_general(a_ref[...], w_ref[...], (((1,), (1,)), ((), ())),
                          preferred_element_type=jnp.float32)
    out_ref[...] = out + b_ref[...]


def kernel(src, tgt, src_table, tgt_table, W_pred, b_pred):
    src_i = src.reshape(-1).astype(jnp.int32)
    tgt_i = tgt.reshape(-1).astype(jnp.int32)

    goff = src_i & ~jnp.int32(7)
    lane = 8 * (jnp.arange(N_SRC, dtype=jnp.int32) % 8) + (src_i & 7)
    oh = (jnp.arange(64, dtype=jnp.int32)[None, :] == lane[:, None]
          ).astype(jnp.float32)

    ctx = pl.pallas_call(
        _gatt_body,
        in_specs=[
            pl.BlockSpec(memory_space=pltpu.SMEM),
            pl.BlockSpec(memory_space=pltpu.SMEM),
            pl.BlockSpec((SRC_VOCAB, D), lambda: (0, 0)),
            pl.BlockSpec((N_SRC, 64), lambda: (0, 0)),
            pl.BlockSpec(memory_space=pl.ANY),
        ],
        out_shape=jax.ShapeDtypeStruct((S_TGT, B, D), jnp.float32),
        scratch_shapes=[
            pltpu.VMEM((N_SRC, D), jnp.float32),
            pltpu.VMEM((N_TGT, D), jnp.float32),
            pltpu.VMEM((64, 64), jnp.float32),
            pltpu.SemaphoreType.DMA,
        ],
    )(goff, tgt_i, src_table, oh, tgt_table)

    a = ctx.reshape(N_TGT, D)
    b2 = b_pred.reshape(1, TGT_VOCAB)
    nv = pl.cdiv(TGT_VOCAB, V_TILE)
    logits = pl.pallas_call(
        _proj_body,
        grid=(nv,),
        in_specs=[
            pl.BlockSpec((N_TGT, D), lambda v: (0, 0)),
            pl.BlockSpec((V_TILE, D), lambda v: (v, 0)),
            pl.BlockSpec((1, V_TILE), lambda v: (0, v)),
        ],
        out_specs=pl.BlockSpec((N_TGT, V_TILE), lambda v: (0, v)),
        out_shape=jax.ShapeDtypeStruct((N_TGT, TGT_VOCAB), jnp.float32),
        compiler_params=pltpu.CompilerParams(
            dimension_semantics=("arbitrary",)),
    )(a, W_pred, b2)
    return logits.reshape(S_TGT, B, TGT_VOCAB)


# speedup vs baseline: 1.4291x; 1.4291x over previous
import jax
import jax.numpy as jnp
from jax import lax
from jax.experimental import pallas as pl
from jax.experimental.pallas import tpu as pltpu

SRC_VOCAB = 100000
TGT_VOCAB = 100000
D = 64
B, S_SRC, S_TGT = 32, 200, 16
N_SRC = B * S_SRC
N_TGT = B * S_TGT
V_TILE = 8192


def _gather_body(sidx_ref, tidx_ref, stab_ref, ttab_ref, se_ref, te_ref, sem):
    def issue_t(i, c):
        pltpu.make_async_copy(ttab_ref.at[pl.ds(tidx_ref[i], 1)],
                              te_ref.at[pl.ds(i, 1)], sem).start()
        return c

    lax.fori_loop(0, N_TGT, issue_t, 0, unroll=8)

    def cp(i, c):
        se_ref[pl.ds(i, 1), :] = stab_ref[pl.ds(sidx_ref[i], 1), :]
        return c

    lax.fori_loop(0, N_SRC, cp, 0, unroll=8)

    pltpu.make_async_copy(ttab_ref.at[pl.ds(0, N_TGT)], te_ref, sem).wait()


def _projattn_body(se_ref, te_ref, w_ref, b_ref, out_ref, ctx_ref):
    @pl.when(pl.program_id(0) == 0)
    def _():
        for b in range(B):
            se_b = se_ref[pl.ds(b * S_SRC, S_SRC), :]
            te_b = te_ref[pl.ds(b * S_TGT, S_TGT), :]
            s = lax.dot_general(te_b, se_b, (((1,), (1,)), ((), ())),
                                preferred_element_type=jnp.float32) * 0.125
            s = s - jnp.max(s, axis=1, keepdims=True)
            e = jnp.exp(s)
            a = e / jnp.sum(e, axis=1, keepdims=True)
            o = lax.dot_general(a, se_b, (((1,), (0,)), ((), ())),
                                preferred_element_type=jnp.float32)
            ctx_ref[:, b, :] = o

    acts = ctx_ref[...].reshape(N_TGT, D)
    out = lax.dot_general(acts, w_ref[...], (((1,), (1,)), ((), ())),
                          preferred_element_type=jnp.float32)
    out_ref[...] = out.reshape(S_TGT, B, -1) + b_ref[...]


def kernel(src, tgt, src_table, tgt_table, W_pred, b_pred):
    src_i = src.reshape(-1).astype(jnp.int32)
    tgt_i = tgt.reshape(-1).astype(jnp.int32)

    se, te = pl.pallas_call(
        _gather_body,
        in_specs=[
            pl.BlockSpec(memory_space=pltpu.SMEM),
            pl.BlockSpec(memory_space=pltpu.SMEM),
            pl.BlockSpec((SRC_VOCAB, D), lambda: (0, 0)),
            pl.BlockSpec(memory_space=pl.ANY),
        ],
        out_shape=[
            jax.ShapeDtypeStruct((N_SRC, D), jnp.float32),
            jax.ShapeDtypeStruct((N_TGT, D), jnp.float32),
        ],
        scratch_shapes=[pltpu.SemaphoreType.DMA],
    )(src_i, tgt_i, src_table, tgt_table)

    b3 = b_pred.reshape(1, 1, TGT_VOCAB)
    nv = pl.cdiv(TGT_VOCAB, V_TILE)
    logits = pl.pallas_call(
        _projattn_body,
        grid=(nv,),
        in_specs=[
            pl.BlockSpec((N_SRC, D), lambda v: (0, 0)),
            pl.BlockSpec((N_TGT, D), lambda v: (0, 0)),
            pl.BlockSpec((V_TILE, D), lambda v: (v, 0)),
            pl.BlockSpec((1, 1, V_TILE), lambda v: (0, 0, v)),
        ],
        out_specs=pl.BlockSpec((S_TGT, B, V_TILE), lambda v: (0, 0, v)),
        out_shape=jax.ShapeDtypeStruct((S_TGT, B, TGT_VOCAB), jnp.float32),
        scratch_shapes=[pltpu.VMEM((S_TGT, B, D), jnp.float32)],
        compiler_params=pltpu.CompilerParams(
            dimension_semantics=("arbitrary",)),
    )(se, te, W_pred, b3)
    return logits
